# Initial kernel scaffold; baseline (speedup 1.0000x reference)
import jax
import jax.numpy as jnp
from jax.experimental import pallas as pl


def kernel(x, weight):
    raise NotImplementedError("write your pallas kernel here")


# speedup vs baseline: 1.0659x; 1.0659x over previous
import math

import jax
import jax.numpy as jnp
from jax import lax
from jax.experimental import pallas as pl
from jax.experimental.pallas import tpu as pltpu

_PI = 3.14159265
_T1 = math.cos(1.0 * _PI / 4.0)
_T2 = math.cos(2.0 * _PI / 4.0)
_T3 = math.cos(3.0 * _PI / 4.0)
_T4 = math.cos(4.0 * _PI / 4.0)


def _angle_linear_body(x_ref, w_ref, cos_ref, phi_ref):
    xf = x_ref[...]
    sx = jnp.sum(xf * xf, axis=1, keepdims=True)
    inv_x = lax.rsqrt(jnp.maximum(sx, 1e-30))
    xlen = sx * inv_x

    wf = w_ref[...]
    sw = jnp.sum(wf * wf, axis=0, keepdims=True)
    inv_w = lax.rsqrt(jnp.maximum(sw, 1e-30))

    dot = jnp.dot(xf.astype(jnp.bfloat16), wf.astype(jnp.bfloat16),
                  preferred_element_type=jnp.float32)
    c = jnp.clip(dot * inv_x * inv_w, -1.0, 1.0)

    c2 = c * c
    cos4 = (8.0 * c2 - 8.0) * c2 + 1.0

    one = jnp.float32(1.0)
    zero = jnp.float32(0.0)
    k = (jnp.where(c <= _T1, one, zero) + jnp.where(c <= _T2, one, zero)
         + jnp.where(c <= _T3, one, zero) + jnp.where(c <= _T4, one, zero))
    sign = 1.0 - 2.0 * jnp.mod(k, 2.0)
    phi = sign * cos4 - 2.0 * k

    cos_ref[...] = c * xlen
    phi_ref[...] = phi * xlen


def kernel(x, weight):
    B, D = x.shape
    D2, N = weight.shape
    assert D == D2

    tn = 1024 if N % 1024 == 0 else min(N, 2048)
    grid = (pl.cdiv(N, tn),)

    cos_t, phi_t = pl.pallas_call(
        _angle_linear_body,
        out_shape=(
            jax.ShapeDtypeStruct((B, N), x.dtype),
            jax.ShapeDtypeStruct((B, N), x.dtype),
        ),
        grid=grid,
        in_specs=[
            pl.BlockSpec((B, D), lambda j: (0, 0)),
            pl.BlockSpec((D, tn), lambda j: (0, j)),
        ],
        out_specs=(
            pl.BlockSpec((B, tn), lambda j: (0, j)),
            pl.BlockSpec((B, tn), lambda j: (0, j)),
        ),
        compiler_params=pltpu.CompilerParams(
            dimension_semantics=("parallel",),
            vmem_limit_bytes=48 << 20,
        ),
    )(x, weight)
    return cos_t, phi_t


# speedup vs baseline: 1.4895x; 1.3974x over previous
import math

import jax
import jax.numpy as jnp
from jax import lax
from jax.experimental import pallas as pl
from jax.experimental.pallas import tpu as pltpu

_PI = 3.14159265
_T1 = math.cos(1.0 * _PI / 4.0)
_T2 = math.cos(2.0 * _PI / 4.0)
_T3 = math.cos(3.0 * _PI / 4.0)


def _angle_linear_body(x_ref, w_ref, cos_ref, phi_ref):
    xf = x_ref[...]
    sx = jnp.sum(xf * xf, axis=1, keepdims=True)
    inv_x = lax.rsqrt(jnp.maximum(sx, 1e-30))
    xlen = sx * inv_x
    xn = (xf * inv_x).astype(jnp.bfloat16)

    wf = w_ref[...]
    sw = jnp.sum(wf * wf, axis=0, keepdims=True)
    inv_w = lax.rsqrt(jnp.maximum(sw, 1e-30))
    wn = (wf * inv_w).astype(jnp.bfloat16)

    dot = jnp.dot(xn, wn, preferred_element_type=jnp.float32)
    c = jnp.clip(dot, -1.0, 1.0)

    c2 = c * c
    p = (8.0 * c2 - 8.0) * c2

    m1 = c <= _T1
    m2 = c <= _T2
    m3 = c <= _T3
    parity = jnp.logical_xor(jnp.logical_xor(m1, m2), m3)
    sp = jnp.where(parity, -p, p)
    qa = jnp.where(m1, jnp.float32(-3.0), jnp.float32(1.0))
    q = jnp.where(m3, qa - 4.0, qa)
    phi = sp + q

    cos_ref[...] = c * xlen
    phi_ref[...] = phi * xlen


def kernel(x, weight):
    B, D = x.shape
    D2, N = weight.shape
    assert D == D2

    tn = 1024 if N % 1024 == 0 else min(N, 2048)
    grid = (pl.cdiv(N, tn),)

    cos_t, phi_t = pl.pallas_call(
        _angle_linear_body,
        out_shape=(
            jax.ShapeDtypeStruct((B, N), x.dtype),
            jax.ShapeDtypeStruct((B, N), x.dtype),
        ),
        grid=grid,
        in_specs=[
            pl.BlockSpec((B, D), lambda j: (0, 0)),
            pl.BlockSpec((D, tn), lambda j: (0, j)),
        ],
        out_specs=(
            pl.BlockSpec((B, tn), lambda j: (0, j)),
            pl.BlockSpec((B, tn), lambda j: (0, j)),
        ),
        compiler_params=pltpu.CompilerParams(
            dimension_semantics=("parallel",),
            vmem_limit_bytes=48 << 20,
        ),
    )(x, weight)
    return cos_t, phi_t


# speedup vs baseline: 1.5683x; 1.0529x over previous
import math

import jax
import jax.numpy as jnp
from jax import lax
from jax.experimental import pallas as pl
from jax.experimental.pallas import tpu as pltpu

_PI = 3.14159265
_T1 = math.cos(1.0 * _PI / 4.0)
_T2 = math.cos(2.0 * _PI / 4.0)
_T3 = math.cos(3.0 * _PI / 4.0)


def _angle_linear_body(x_ref, w_ref, cos_ref, phi_ref):
    xf = x_ref[...]
    sx = jnp.sum(xf * xf, axis=1, keepdims=True)
    inv_x = lax.rsqrt(jnp.maximum(sx, 1e-30))
    xlen = sx * inv_x
    xn = (xf * inv_x).astype(jnp.bfloat16)

    wf = w_ref[...]
    sw = jnp.sum(wf * wf, axis=0, keepdims=True)
    inv_w = lax.rsqrt(jnp.maximum(sw, 1e-30))
    wn = (wf * inv_w).astype(jnp.bfloat16)

    dot = jnp.dot(xn, wn, preferred_element_type=jnp.float32)
    c = jnp.clip(dot, -1.0, 1.0)

    c2 = c * c
    p = (8.0 * c2 - 8.0) * c2

    m1 = c <= _T1
    m2 = c <= _T2
    m3 = c <= _T3
    parity = jnp.logical_xor(jnp.logical_xor(m1, m2), m3)
    sp = jnp.where(parity, -p, p)
    qa = jnp.where(m1, jnp.float32(-3.0), jnp.float32(1.0))
    q = jnp.where(m3, qa - 4.0, qa)
    phi = sp + q

    cos_ref[...] = c * xlen
    phi_ref[...] = phi * xlen


def kernel(x, weight):
    B, D = x.shape
    D2, N = weight.shape
    assert D == D2

    tn = 2048 if N % 2048 == 0 else min(N, 2048)
    grid = (pl.cdiv(N, tn),)

    cos_t, phi_t = pl.pallas_call(
        _angle_linear_body,
        out_shape=(
            jax.ShapeDtypeStruct((B, N), x.dtype),
            jax.ShapeDtypeStruct((B, N), x.dtype),
        ),
        grid=grid,
        in_specs=[
            pl.BlockSpec((B, D), lambda j: (0, 0)),
            pl.BlockSpec((D, tn), lambda j: (0, j)),
        ],
        out_specs=(
            pl.BlockSpec((B, tn), lambda j: (0, j)),
            pl.BlockSpec((B, tn), lambda j: (0, j)),
        ),
        compiler_params=pltpu.CompilerParams(
            dimension_semantics=("parallel",),
            vmem_limit_bytes=48 << 20,
        ),
    )(x, weight)
    return cos_t, phi_t
